# baseline (device time: 64286 ns/iter reference)
import jax
import jax.numpy as jnp
from jax import lax
from jax.experimental import pallas as pl
from jax.experimental.pallas import tpu as pltpu

N_DEV = 32
BN = 1024

F8 = jnp.float8_e4m3fn


def kernel(x, w_mat, scale_x, scale_w):
    M, k_per = x.shape
    K, N = w_mat.shape
    m_per = M // N_DEV
    n_steps = N // BN

    def body(x_ref, w_ref, sx_ref, sw_ref, out_ref, x8_ref, xt_ref,
             send_sems, recv_sems):
        j = pl.program_id(0)
        my = lax.axis_index("i")

        @pl.when(j == 0)
        def _comm():
            barrier_sem = pltpu.get_barrier_semaphore()
            for s in range(1, N_DEV):
                peer = lax.rem(my + s, N_DEV)
                pl.semaphore_signal(
                    barrier_sem, inc=1,
                    device_id=(peer,), device_id_type=pl.DeviceIdType.MESH,
                )
            pl.semaphore_wait(barrier_sem, N_DEV - 1)

            x8_ref[:, :] = x_ref[:, :].astype(F8)
            xt_ref[:, pl.ds(my * k_per, k_per)] = x8_ref[pl.ds(my * m_per, m_per), :]

            sends = []
            for s in range(1, N_DEV):
                dst = lax.rem(my + s, N_DEV)
                rdma = pltpu.make_async_remote_copy(
                    src_ref=x8_ref.at[pl.ds(dst * m_per, m_per), :],
                    dst_ref=xt_ref.at[:, pl.ds(my * k_per, k_per)],
                    send_sem=send_sems.at[s],
                    recv_sem=recv_sems.at[my],
                    device_id=(dst,),
                    device_id_type=pl.DeviceIdType.MESH,
                )
                rdma.start()
                sends.append(rdma)

            for s in range(1, N_DEV):
                src = lax.rem(my + s, N_DEV)
                recv = pltpu.make_async_remote_copy(
                    src_ref=x8_ref.at[pl.ds(0, m_per), :],
                    dst_ref=xt_ref.at[:, pl.ds(src * k_per, k_per)],
                    send_sem=send_sems.at[s],
                    recv_sem=recv_sems.at[src],
                    device_id=(src,),
                    device_id_type=pl.DeviceIdType.MESH,
                )
                recv.wait_recv()

            for rdma in sends:
                rdma.wait_send()

        wb = w_ref[:, :].astype(jnp.bfloat16)
        acc = lax.dot_general(
            xt_ref[:, :].astype(jnp.bfloat16), wb,
            (((1,), (0,)), ((), ())),
            preferred_element_type=jnp.float32,
        )
        y = acc * (sx_ref[0] * sw_ref[0])
        out_ref[:, :] = y * jax.nn.sigmoid(y)

    return pl.pallas_call(
        body,
        grid=(n_steps,),
        out_shape=jax.ShapeDtypeStruct((m_per, N), jnp.float32),
        in_specs=[
            pl.BlockSpec((M, k_per), lambda j: (0, 0)),
            pl.BlockSpec((K, BN), lambda j: (0, j)),
            pl.BlockSpec(memory_space=pltpu.SMEM),
            pl.BlockSpec(memory_space=pltpu.SMEM),
        ],
        out_specs=pl.BlockSpec((m_per, BN), lambda j: (0, j)),
        scratch_shapes=[
            pltpu.VMEM((M, k_per), F8),
            pltpu.VMEM((m_per, K), F8),
            pltpu.SemaphoreType.DMA((N_DEV,)),
            pltpu.SemaphoreType.DMA((N_DEV,)),
        ],
        compiler_params=pltpu.CompilerParams(
            collective_id=0,
            vmem_limit_bytes=60 * 1024 * 1024,
        ),
    )(x, w_mat, scale_x, scale_w)


# device time: 63412 ns/iter; 1.0138x vs baseline; 1.0138x over previous
import jax
import jax.numpy as jnp
from jax import lax
from jax.experimental import pallas as pl
from jax.experimental.pallas import tpu as pltpu

N_DEV = 32
BK = 512

F8 = jnp.float8_e4m3fn
BF16 = jnp.bfloat16


def kernel(x, w_mat, scale_x, scale_w):
    M, k_per = x.shape
    K, N = w_mat.shape
    m_per = M // N_DEV
    n_steps = K // BK
    src_per_blk = BK // k_per

    def body(x_ref, w_ref, sx_ref, sw_ref, out_ref, x8_ref, xt_ref,
             send_sems, recv_sems):
        j = pl.program_id(0)
        my = lax.axis_index("i")

        @pl.when(j == 0)
        def _comm():
            barrier_sem = pltpu.get_barrier_semaphore()
            for s in range(1, N_DEV):
                peer = lax.rem(my + s, N_DEV)
                pl.semaphore_signal(
                    barrier_sem, inc=1,
                    device_id=(peer,), device_id_type=pl.DeviceIdType.MESH,
                )
            pl.semaphore_wait(barrier_sem, N_DEV - 1)

            x8_ref[:, :] = x_ref[:, :].astype(F8)
            xt_ref[:, pl.ds(my * k_per, k_per)] = x8_ref[pl.ds(my * m_per, m_per), :]

            for s in range(1, N_DEV):
                dst = lax.rem(my + s, N_DEV)
                rdma = pltpu.make_async_remote_copy(
                    src_ref=x8_ref.at[pl.ds(dst * m_per, m_per), :],
                    dst_ref=xt_ref.at[:, pl.ds(my * k_per, k_per)],
                    send_sem=send_sems.at[s],
                    recv_sem=recv_sems.at[my],
                    device_id=(dst,),
                    device_id_type=pl.DeviceIdType.MESH,
                )
                rdma.start()

        for t in range(src_per_blk):
            src = j * src_per_blk + t

            @pl.when(src != my)
            def _wait(src=src):
                recv = pltpu.make_async_remote_copy(
                    src_ref=x8_ref.at[pl.ds(0, m_per), :],
                    dst_ref=xt_ref.at[:, pl.ds(src * k_per, k_per)],
                    send_sem=send_sems.at[0],
                    recv_sem=recv_sems.at[src],
                    device_id=(src,),
                    device_id_type=pl.DeviceIdType.MESH,
                )
                recv.wait_recv()

        xb = xt_ref[:, pl.ds(j * BK, BK)].astype(BF16)
        wb = w_ref[:, :].astype(BF16)
        part = lax.dot_general(
            xb, wb, (((1,), (0,)), ((), ())),
            preferred_element_type=jnp.float32,
        )

        @pl.when(j == 0)
        def _init():
            out_ref[:, :] = part

        @pl.when(j > 0)
        def _acc():
            out_ref[:, :] += part

        @pl.when(j == n_steps - 1)
        def _epilogue():
            y = out_ref[:, :] * (sx_ref[0] * sw_ref[0])
            out_ref[:, :] = y * jax.nn.sigmoid(y)

            for s in range(1, N_DEV):
                dst = lax.rem(my + s, N_DEV)
                snd = pltpu.make_async_remote_copy(
                    src_ref=x8_ref.at[pl.ds(dst * m_per, m_per), :],
                    dst_ref=xt_ref.at[:, pl.ds(my * k_per, k_per)],
                    send_sem=send_sems.at[s],
                    recv_sem=recv_sems.at[my],
                    device_id=(dst,),
                    device_id_type=pl.DeviceIdType.MESH,
                )
                snd.wait_send()

    return pl.pallas_call(
        body,
        grid=(n_steps,),
        out_shape=jax.ShapeDtypeStruct((m_per, N), jnp.float32),
        in_specs=[
            pl.BlockSpec((M, k_per), lambda j: (0, 0)),
            pl.BlockSpec((BK, N), lambda j: (j, 0)),
            pl.BlockSpec(memory_space=pltpu.SMEM),
            pl.BlockSpec(memory_space=pltpu.SMEM),
        ],
        out_specs=pl.BlockSpec((m_per, N), lambda j: (0, 0)),
        scratch_shapes=[
            pltpu.VMEM((M, k_per), F8),
            pltpu.VMEM((m_per, K), F8),
            pltpu.SemaphoreType.DMA((N_DEV,)),
            pltpu.SemaphoreType.DMA((N_DEV,)),
        ],
        compiler_params=pltpu.CompilerParams(
            collective_id=0,
            vmem_limit_bytes=60 * 1024 * 1024,
        ),
    )(x, w_mat, scale_x, scale_w)


# device time: 62492 ns/iter; 1.0287x vs baseline; 1.0147x over previous
import os

import jax
import jax.numpy as jnp
from jax import lax
from jax.experimental import pallas as pl
from jax.experimental.pallas import tpu as pltpu

N_DEV = 32
BK = 512
_DISABLE_COMM = os.environ.get("KERNEL_DISABLE_COMM") == "1"

F8 = jnp.float8_e4m3fn
BF16 = jnp.bfloat16


def kernel(x, w_mat, scale_x, scale_w):
    M, k_per = x.shape
    K, N = w_mat.shape
    m_per = M // N_DEV
    n_steps = K // BK
    src_per_blk = BK // k_per

    my_sm = lax.axis_index("i")
    my_blk_sm = my_sm // src_per_blk
    blk_order = (my_blk_sm + jnp.arange(n_steps, dtype=jnp.int32)) % n_steps

    def body(idx_ref, x_ref, w_ref, sx_ref, sw_ref, out_ref, x8_ref, xt_ref,
             send_sems, recv_sems):
        j = pl.program_id(0)
        my = lax.axis_index("i")
        my_blk = lax.div(my, src_per_blk)
        kb = idx_ref[j]

        @pl.when(j == 0)
        def _comm():
            x8_ref[:, :] = x_ref[:, :].astype(F8)
            xt_ref[:, pl.ds(my * k_per, k_per)] = x8_ref[pl.ds(my * m_per, m_per), :]

            if not _DISABLE_COMM:
                barrier_sem = pltpu.get_barrier_semaphore()
                for s in range(1, N_DEV):
                    peer = lax.rem(my + s, N_DEV)
                    pl.semaphore_signal(
                        barrier_sem, inc=1,
                        device_id=(peer,), device_id_type=pl.DeviceIdType.MESH,
                    )
                pl.semaphore_wait(barrier_sem, N_DEV - 1)

                for jj in range(n_steps):
                    dblk = lax.rem(my_blk - jj + n_steps, n_steps)
                    for r in range(src_per_blk):
                        dst = dblk * src_per_blk + r
                        slot = jj * src_per_blk + r

                        @pl.when(dst != my)
                        def _send(dst=dst, slot=slot):
                            rdma = pltpu.make_async_remote_copy(
                                src_ref=x8_ref.at[pl.ds(dst * m_per, m_per), :],
                                dst_ref=xt_ref.at[:, pl.ds(my * k_per, k_per)],
                                send_sem=send_sems.at[slot],
                                recv_sem=recv_sems.at[my],
                                device_id=(dst,),
                                device_id_type=pl.DeviceIdType.MESH,
                            )
                            rdma.start()

        if not _DISABLE_COMM:
            for t in range(src_per_blk):
                src = kb * src_per_blk + t

                @pl.when(src != my)
                def _wait(src=src):
                    recv = pltpu.make_async_remote_copy(
                        src_ref=x8_ref.at[pl.ds(0, m_per), :],
                        dst_ref=xt_ref.at[:, pl.ds(src * k_per, k_per)],
                        send_sem=send_sems.at[0],
                        recv_sem=recv_sems.at[src],
                        device_id=(src,),
                        device_id_type=pl.DeviceIdType.MESH,
                    )
                    recv.wait_recv()

        xb = xt_ref[:, pl.ds(kb * BK, BK)].astype(BF16)
        wb = w_ref[:, :].astype(BF16)
        part = lax.dot_general(
            xb, wb, (((1,), (0,)), ((), ())),
            preferred_element_type=jnp.float32,
        )

        @pl.when(j == 0)
        def _init():
            out_ref[:, :] = part

        @pl.when(j > 0)
        def _acc():
            out_ref[:, :] += part

        @pl.when(j == n_steps - 1)
        def _epilogue():
            y = out_ref[:, :] * (sx_ref[0] * sw_ref[0])
            out_ref[:, :] = y * jax.nn.sigmoid(y)

            if not _DISABLE_COMM:
                for jj in range(n_steps):
                    dblk = lax.rem(my_blk - jj + n_steps, n_steps)
                    for r in range(src_per_blk):
                        dst = dblk * src_per_blk + r
                        slot = jj * src_per_blk + r

                        @pl.when(dst != my)
                        def _drain(dst=dst, slot=slot):
                            snd = pltpu.make_async_remote_copy(
                                src_ref=x8_ref.at[pl.ds(dst * m_per, m_per), :],
                                dst_ref=xt_ref.at[:, pl.ds(my * k_per, k_per)],
                                send_sem=send_sems.at[slot],
                                recv_sem=recv_sems.at[my],
                                device_id=(dst,),
                                device_id_type=pl.DeviceIdType.MESH,
                            )
                            snd.wait_send()

    grid_spec = pltpu.PrefetchScalarGridSpec(
        num_scalar_prefetch=1,
        grid=(n_steps,),
        in_specs=[
            pl.BlockSpec((M, k_per), lambda j, idx: (0, 0)),
            pl.BlockSpec((BK, N), lambda j, idx: (idx[j], 0)),
            pl.BlockSpec(memory_space=pltpu.SMEM),
            pl.BlockSpec(memory_space=pltpu.SMEM),
        ],
        out_specs=pl.BlockSpec((m_per, N), lambda j, idx: (0, 0)),
        scratch_shapes=[
            pltpu.VMEM((M, k_per), F8),
            pltpu.VMEM((m_per, K), F8),
            pltpu.SemaphoreType.DMA((N_DEV,)),
            pltpu.SemaphoreType.DMA((N_DEV,)),
        ],
    )

    return pl.pallas_call(
        body,
        grid_spec=grid_spec,
        out_shape=jax.ShapeDtypeStruct((m_per, N), jnp.float32),
        compiler_params=pltpu.CompilerParams(
            collective_id=None if _DISABLE_COMM else 0,
            vmem_limit_bytes=60 * 1024 * 1024,
        ),
    )(blk_order, x, w_mat, scale_x, scale_w)


# device time: 60565 ns/iter; 1.0614x vs baseline; 1.0318x over previous
import os

import jax
import jax.numpy as jnp
from jax import lax
from jax.experimental import pallas as pl
from jax.experimental.pallas import tpu as pltpu

N_DEV = 32
BK = 512
_PROBE = os.environ.get("KERNEL_PROBE", "")
_DISABLE_COMM = _PROBE == "nocomm"

F8 = jnp.float8_e4m3fn
BF16 = jnp.bfloat16


def kernel(x, w_mat, scale_x, scale_w):
    M, k_per = x.shape
    K, N = w_mat.shape
    m_per = M // N_DEV
    n_steps = K // BK
    src_per_blk = BK // k_per

    my_sm = lax.axis_index("i")
    blk_order = (my_sm // src_per_blk
                 + jnp.arange(n_steps, dtype=jnp.int32)) % n_steps

    def body(idx_ref, x_ref, w_ref, sx_ref, sw_ref, out_ref, x8_ref, xt_ref,
             send_sems, recv_sems, credit_sems):
        j = pl.program_id(0)
        my = lax.axis_index("i")
        my_blk = lax.div(my, src_per_blk)
        kb = idx_ref[j]

        @pl.when(j == 0)
        def _comm():
            if not _DISABLE_COMM:
                for s in range(1, N_DEV):
                    peer = lax.rem(my + s, N_DEV)
                    pl.semaphore_signal(
                        credit_sems.at[my], inc=1,
                        device_id=(peer,), device_id_type=pl.DeviceIdType.MESH,
                    )

            x8_ref[:, :] = x_ref[:, :].astype(F8)
            xt_ref[:, pl.ds(my * k_per, k_per)] = x8_ref[pl.ds(my * m_per, m_per), :]
            if _DISABLE_COMM:
                for t in range(N_DEV):
                    xt_ref[:, pl.ds(t * k_per, k_per)] = x8_ref[pl.ds(t * m_per, m_per), :]

            if not _DISABLE_COMM:
                for jj in range(n_steps):
                    dblk = lax.rem(my_blk - jj + n_steps, n_steps)
                    for r in range(src_per_blk):
                        dst = dblk * src_per_blk + r
                        slot = jj * src_per_blk + r

                        @pl.when(dst != my)
                        def _send(dst=dst, slot=slot):
                            pl.semaphore_wait(credit_sems.at[dst], 1)
                            rdma = pltpu.make_async_remote_copy(
                                src_ref=x8_ref.at[pl.ds(dst * m_per, m_per), :],
                                dst_ref=xt_ref.at[:, pl.ds(my * k_per, k_per)],
                                send_sem=send_sems.at[slot],
                                recv_sem=recv_sems.at[my],
                                device_id=(dst,),
                                device_id_type=pl.DeviceIdType.MESH,
                            )
                            rdma.start()

        if not _DISABLE_COMM:
            for t in range(src_per_blk):
                src = kb * src_per_blk + t

                @pl.when(src != my)
                def _wait(src=src):
                    recv = pltpu.make_async_remote_copy(
                        src_ref=x8_ref.at[pl.ds(0, m_per), :],
                        dst_ref=xt_ref.at[:, pl.ds(src * k_per, k_per)],
                        send_sem=send_sems.at[0],
                        recv_sem=recv_sems.at[src],
                        device_id=(src,),
                        device_id_type=pl.DeviceIdType.MESH,
                    )
                    recv.wait_recv()

        xb = xt_ref[:, pl.ds(kb * BK, BK)].astype(BF16)
        wb = w_ref[:, :].astype(BF16)
        part = lax.dot_general(
            xb, wb, (((1,), (0,)), ((), ())),
            preferred_element_type=jnp.float32,
        )

        @pl.when(j == 0)
        def _init():
            out_ref[:, :] = part

        @pl.when(j > 0)
        def _acc():
            out_ref[:, :] += part

        @pl.when(j == n_steps - 1)
        def _epilogue():
            y = out_ref[:, :] * (sx_ref[0] * sw_ref[0])
            out_ref[:, :] = y * jax.nn.sigmoid(y)

            if not _DISABLE_COMM:
                for jj in range(n_steps):
                    dblk = lax.rem(my_blk - jj + n_steps, n_steps)
                    for r in range(src_per_blk):
                        dst = dblk * src_per_blk + r
                        slot = jj * src_per_blk + r

                        @pl.when(dst != my)
                        def _drain(dst=dst, slot=slot):
                            snd = pltpu.make_async_remote_copy(
                                src_ref=x8_ref.at[pl.ds(dst * m_per, m_per), :],
                                dst_ref=xt_ref.at[:, pl.ds(my * k_per, k_per)],
                                send_sem=send_sems.at[slot],
                                recv_sem=recv_sems.at[my],
                                device_id=(dst,),
                                device_id_type=pl.DeviceIdType.MESH,
                            )
                            snd.wait_send()

    grid_spec = pltpu.PrefetchScalarGridSpec(
        num_scalar_prefetch=1,
        grid=(n_steps,),
        in_specs=[
            pl.BlockSpec((M, k_per), lambda j, idx: (0, 0)),
            pl.BlockSpec((BK, N), lambda j, idx: (idx[j], 0)),
            pl.BlockSpec(memory_space=pltpu.SMEM),
            pl.BlockSpec(memory_space=pltpu.SMEM),
        ],
        out_specs=pl.BlockSpec((m_per, N), lambda j, idx: (0, 0)),
        scratch_shapes=[
            pltpu.VMEM((M, k_per), F8),
            pltpu.VMEM((m_per, K), F8),
            pltpu.SemaphoreType.DMA((N_DEV,)),
            pltpu.SemaphoreType.DMA((N_DEV,)),
            pltpu.SemaphoreType.REGULAR((N_DEV,)),
        ],
    )

    return pl.pallas_call(
        body,
        grid_spec=grid_spec,
        out_shape=jax.ShapeDtypeStruct((m_per, N), jnp.float32),
        compiler_params=pltpu.CompilerParams(
            skip_device_barrier=not _DISABLE_COMM,
            vmem_limit_bytes=60 * 1024 * 1024,
        ),
    )(blk_order, x, w_mat, scale_x, scale_w)


# device time: 60204 ns/iter; 1.0678x vs baseline; 1.0060x over previous
import os

import jax
import jax.numpy as jnp
from jax import lax
from jax.experimental import pallas as pl
from jax.experimental.pallas import tpu as pltpu

N_DEV = 32
BK = 512
_PROBE = os.environ.get("KERNEL_PROBE", "")
_DISABLE_COMM = _PROBE == "nocomm"

F8 = jnp.float8_e4m3fn
BF16 = jnp.bfloat16


def kernel(x, w_mat, scale_x, scale_w):
    M, k_per = x.shape
    K, N = w_mat.shape
    m_per = M // N_DEV
    n_steps = K // BK
    src_per_blk = BK // k_per

    my_sm = lax.axis_index("i")
    blk_order = (my_sm // src_per_blk
                 + jnp.arange(n_steps, dtype=jnp.int32)) % n_steps

    def body(idx_ref, x_ref, w_ref, sx_ref, sw_ref, out_ref, x8_ref, xt_ref,
             send_sems, recv_sems, credit_sems):
        j = pl.program_id(0)
        my = lax.axis_index("i")
        my_blk = lax.div(my, src_per_blk)
        kb = idx_ref[j]

        @pl.when(j == 0)
        def _comm():
            if not _DISABLE_COMM:
                for s in range(1, N_DEV):
                    peer = lax.rem(my + s, N_DEV)
                    pl.semaphore_signal(
                        credit_sems.at[my], inc=1,
                        device_id=(peer,), device_id_type=pl.DeviceIdType.MESH,
                    )

            x8_ref[:, :] = x_ref[:, :].astype(F8)
            xt_ref[:, pl.ds(my * k_per, k_per)] = x8_ref[pl.ds(my * m_per, m_per), :]
            if _DISABLE_COMM:
                for t in range(N_DEV):
                    xt_ref[:, pl.ds(t * k_per, k_per)] = x8_ref[pl.ds(t * m_per, m_per), :]

            if not _DISABLE_COMM:
                for jj in range(n_steps):
                    dblk = lax.rem(my_blk - jj + n_steps, n_steps)
                    for r in range(src_per_blk):
                        dst = dblk * src_per_blk + r
                        slot = jj * src_per_blk + r

                        @pl.when(dst != my)
                        def _send(dst=dst, slot=slot):
                            pl.semaphore_wait(credit_sems.at[dst], 1)
                            rdma = pltpu.make_async_remote_copy(
                                src_ref=x8_ref.at[pl.ds(dst * m_per, m_per), :],
                                dst_ref=xt_ref.at[:, pl.ds(my * k_per, k_per)],
                                send_sem=send_sems.at[slot],
                                recv_sem=recv_sems.at[my],
                                device_id=(dst,),
                                device_id_type=pl.DeviceIdType.MESH,
                            )
                            rdma.start()

        if not _DISABLE_COMM:
            for t in range(src_per_blk):
                src = kb * src_per_blk + t

                @pl.when(src != my)
                def _wait(src=src):
                    recv = pltpu.make_async_remote_copy(
                        src_ref=x8_ref.at[pl.ds(0, m_per), :],
                        dst_ref=xt_ref.at[:, pl.ds(src * k_per, k_per)],
                        send_sem=send_sems.at[0],
                        recv_sem=recv_sems.at[src],
                        device_id=(src,),
                        device_id_type=pl.DeviceIdType.MESH,
                    )
                    recv.wait_recv()

        xb = xt_ref[:, pl.ds(kb * BK, BK)]
        wb = w_ref[:, :].astype(F8)
        part = lax.dot_general(
            xb, wb, (((1,), (0,)), ((), ())),
            preferred_element_type=jnp.float32,
        )

        @pl.when(j == 0)
        def _init():
            out_ref[:, :] = part

        @pl.when(j > 0)
        def _acc():
            out_ref[:, :] += part

        @pl.when(j == n_steps - 1)
        def _epilogue():
            y = out_ref[:, :] * (sx_ref[0] * sw_ref[0])
            out_ref[:, :] = y * jax.nn.sigmoid(y)

            if not _DISABLE_COMM:
                for jj in range(n_steps):
                    dblk = lax.rem(my_blk - jj + n_steps, n_steps)
                    for r in range(src_per_blk):
                        dst = dblk * src_per_blk + r
                        slot = jj * src_per_blk + r

                        @pl.when(dst != my)
                        def _drain(dst=dst, slot=slot):
                            snd = pltpu.make_async_remote_copy(
                                src_ref=x8_ref.at[pl.ds(dst * m_per, m_per), :],
                                dst_ref=xt_ref.at[:, pl.ds(my * k_per, k_per)],
                                send_sem=send_sems.at[slot],
                                recv_sem=recv_sems.at[my],
                                device_id=(dst,),
                                device_id_type=pl.DeviceIdType.MESH,
                            )
                            snd.wait_send()

    grid_spec = pltpu.PrefetchScalarGridSpec(
        num_scalar_prefetch=1,
        grid=(n_steps,),
        in_specs=[
            pl.BlockSpec((M, k_per), lambda j, idx: (0, 0)),
            pl.BlockSpec((BK, N), lambda j, idx: (idx[j], 0)),
            pl.BlockSpec(memory_space=pltpu.SMEM),
            pl.BlockSpec(memory_space=pltpu.SMEM),
        ],
        out_specs=pl.BlockSpec((m_per, N), lambda j, idx: (0, 0)),
        scratch_shapes=[
            pltpu.VMEM((M, k_per), F8),
            pltpu.VMEM((m_per, K), F8),
            pltpu.SemaphoreType.DMA((N_DEV,)),
            pltpu.SemaphoreType.DMA((N_DEV,)),
            pltpu.SemaphoreType.REGULAR((N_DEV,)),
        ],
    )

    return pl.pallas_call(
        body,
        grid_spec=grid_spec,
        out_shape=jax.ShapeDtypeStruct((m_per, N), jnp.float32),
        compiler_params=pltpu.CompilerParams(
            skip_device_barrier=not _DISABLE_COMM,
            vmem_limit_bytes=60 * 1024 * 1024,
        ),
    )(blk_order, x, w_mat, scale_x, scale_w)


# device time: 60089 ns/iter; 1.0698x vs baseline; 1.0019x over previous
import os

import jax
import jax.numpy as jnp
from jax import lax
from jax.experimental import pallas as pl
from jax.experimental.pallas import tpu as pltpu

N_DEV = 32
BK = 256
_PROBE = os.environ.get("KERNEL_PROBE", "")
_DISABLE_COMM = _PROBE == "nocomm"

F8 = jnp.float8_e4m3fn
BF16 = jnp.bfloat16


def kernel(x, w_mat, scale_x, scale_w):
    M, k_per = x.shape
    K, N = w_mat.shape
    m_per = M // N_DEV
    n_steps = K // BK
    src_per_blk = BK // k_per

    my_sm = lax.axis_index("i")
    blk_order = (my_sm // src_per_blk
                 + jnp.arange(n_steps, dtype=jnp.int32)) % n_steps

    def body(idx_ref, x_ref, w_ref, sx_ref, sw_ref, out_ref, x8_ref, xt_ref,
             send_sems, recv_sems, credit_sems):
        j = pl.program_id(0)
        my = lax.axis_index("i")
        my_blk = lax.div(my, src_per_blk)
        kb = idx_ref[j]

        @pl.when(j == 0)
        def _comm():
            if not _DISABLE_COMM:
                for s in range(1, N_DEV):
                    peer = lax.rem(my + s, N_DEV)
                    pl.semaphore_signal(
                        credit_sems.at[my], inc=1,
                        device_id=(peer,), device_id_type=pl.DeviceIdType.MESH,
                    )

            x8_ref[:, :] = x_ref[:, :].astype(F8)
            xt_ref[:, pl.ds(my * k_per, k_per)] = x8_ref[pl.ds(my * m_per, m_per), :]
            if _DISABLE_COMM:
                for t in range(N_DEV):
                    xt_ref[:, pl.ds(t * k_per, k_per)] = x8_ref[pl.ds(t * m_per, m_per), :]

            if not _DISABLE_COMM:
                for jj in range(n_steps):
                    dblk = lax.rem(my_blk - jj + n_steps, n_steps)
                    for r in range(src_per_blk):
                        dst = dblk * src_per_blk + r
                        slot = jj * src_per_blk + r

                        @pl.when(dst != my)
                        def _send(dst=dst, slot=slot):
                            pl.semaphore_wait(credit_sems.at[dst], 1)
                            rdma = pltpu.make_async_remote_copy(
                                src_ref=x8_ref.at[pl.ds(dst * m_per, m_per), :],
                                dst_ref=xt_ref.at[:, pl.ds(my * k_per, k_per)],
                                send_sem=send_sems.at[slot],
                                recv_sem=recv_sems.at[my],
                                device_id=(dst,),
                                device_id_type=pl.DeviceIdType.MESH,
                            )
                            rdma.start()

        if not _DISABLE_COMM:
            for t in range(src_per_blk):
                src = kb * src_per_blk + t

                @pl.when(src != my)
                def _wait(src=src):
                    recv = pltpu.make_async_remote_copy(
                        src_ref=x8_ref.at[pl.ds(0, m_per), :],
                        dst_ref=xt_ref.at[:, pl.ds(src * k_per, k_per)],
                        send_sem=send_sems.at[0],
                        recv_sem=recv_sems.at[src],
                        device_id=(src,),
                        device_id_type=pl.DeviceIdType.MESH,
                    )
                    recv.wait_recv()

        xb = xt_ref[:, pl.ds(kb * BK, BK)]
        wb = w_ref[:, :].astype(F8)
        part = lax.dot_general(
            xb, wb, (((1,), (0,)), ((), ())),
            preferred_element_type=jnp.float32,
        )

        @pl.when(j == 0)
        def _init():
            out_ref[:, :] = part

        @pl.when(j > 0)
        def _acc():
            out_ref[:, :] += part

        @pl.when(j == n_steps - 1)
        def _epilogue():
            y = out_ref[:, :] * (sx_ref[0] * sw_ref[0])
            out_ref[:, :] = y * jax.nn.sigmoid(y)

            if not _DISABLE_COMM:
                for jj in range(n_steps):
                    dblk = lax.rem(my_blk - jj + n_steps, n_steps)
                    for r in range(src_per_blk):
                        dst = dblk * src_per_blk + r
                        slot = jj * src_per_blk + r

                        @pl.when(dst != my)
                        def _drain(dst=dst, slot=slot):
                            snd = pltpu.make_async_remote_copy(
                                src_ref=x8_ref.at[pl.ds(dst * m_per, m_per), :],
                                dst_ref=xt_ref.at[:, pl.ds(my * k_per, k_per)],
                                send_sem=send_sems.at[slot],
                                recv_sem=recv_sems.at[my],
                                device_id=(dst,),
                                device_id_type=pl.DeviceIdType.MESH,
                            )
                            snd.wait_send()

    grid_spec = pltpu.PrefetchScalarGridSpec(
        num_scalar_prefetch=1,
        grid=(n_steps,),
        in_specs=[
            pl.BlockSpec((M, k_per), lambda j, idx: (0, 0)),
            pl.BlockSpec((BK, N), lambda j, idx: (idx[j], 0)),
            pl.BlockSpec(memory_space=pltpu.SMEM),
            pl.BlockSpec(memory_space=pltpu.SMEM),
        ],
        out_specs=pl.BlockSpec((m_per, N), lambda j, idx: (0, 0)),
        scratch_shapes=[
            pltpu.VMEM((M, k_per), F8),
            pltpu.VMEM((m_per, K), F8),
            pltpu.SemaphoreType.DMA((N_DEV,)),
            pltpu.SemaphoreType.DMA((N_DEV,)),
            pltpu.SemaphoreType.REGULAR((N_DEV,)),
        ],
    )

    return pl.pallas_call(
        body,
        grid_spec=grid_spec,
        out_shape=jax.ShapeDtypeStruct((m_per, N), jnp.float32),
        compiler_params=pltpu.CompilerParams(
            skip_device_barrier=not _DISABLE_COMM,
            vmem_limit_bytes=60 * 1024 * 1024,
        ),
    )(blk_order, x, w_mat, scale_x, scale_w)
